# Initial kernel scaffold; baseline (speedup 1.0000x reference)
import jax
import jax.numpy as jnp
from jax.experimental import pallas as pl


def kernel(inputs, table):
    raise NotImplementedError("write your pallas kernel here")


# speedup vs baseline: 2.2359x; 2.2359x over previous
import functools
import math

import jax
import jax.numpy as jnp
from jax import lax
from jax.experimental import pallas as pl
from jax.experimental.pallas import tpu as pltpu
from jax.experimental.pallas import tpu_sc as plsc

D_MODEL = 768
VOCAB = 100000
SCALE = 1.0 / math.sqrt(D_MODEL)

NC, NS, LANES = 2, 16, 16
NW = NC * NS
N_ROWS = 2048 * 4
R_PER_W = N_ROWS // NW
CHUNK = 32
N_CHUNKS = R_PER_W // CHUNK
NBUF = 3
PE_PER_W = R_PER_W // 4
D_VECS = D_MODEL // LANES


def _pe_table(n_pos):
    pos = jnp.arange(n_pos, dtype=jnp.float32)[:, None]
    div = jnp.exp(jnp.arange(0, D_MODEL, 2, dtype=jnp.float32)
                  * (-math.log(35000.0) / D_MODEL))
    ang = pos * div[None, :]
    return jnp.stack([jnp.sin(ang), jnp.cos(ang)], axis=-1).reshape(n_pos, D_MODEL)


def _sc_body(table_hbm, idx_hbm, pe_hbm, out_hbm,
             idx_v, coef_v, pe_v, buf0, buf1, buf2,
             g0, g1, g2, o0, o1, o2):
    bufs = (buf0, buf1, buf2)
    gsems = (g0, g1, g2)
    osems = (o0, o1, o2)

    wid = lax.axis_index("s") * NC + lax.axis_index("c")
    base = wid * R_PER_W

    pltpu.sync_copy(idx_hbm.at[pl.ds(base, R_PER_W)], idx_v)
    pltpu.sync_copy(pe_hbm.at[pl.ds(wid * PE_PER_W, PE_PER_W)], pe_v)

    for t in range(R_PER_W // LANES):
        iv = idx_v[pl.ds(t * LANES, LANES)]
        coef_v[pl.ds(t * LANES, LANES)] = jnp.where(
            iv != 0, jnp.float32(SCALE), jnp.float32(0.0))

    def issue_gather(k):
        return pltpu.async_copy(
            table_hbm.at[idx_v.at[pl.ds(k * CHUNK, CHUNK)]],
            bufs[k % NBUF], gsems[k % NBUF])

    def issue_out(k):
        return pltpu.async_copy(
            bufs[k % NBUF], out_hbm.at[pl.ds(base + k * CHUNK, CHUNK)],
            osems[k % NBUF])

    pending_g, pending_o = {}, {}
    for k in range(min(NBUF, N_CHUNKS)):
        pending_g[k] = issue_gather(k)

    for m in range(N_CHUNKS):
        if m >= NBUF - 1 and m + 1 < N_CHUNKS:
            pending_o.pop(m + 1 - NBUF).wait()
            pending_g[m + 1] = issue_gather(m + 1)

        pending_g.pop(m).wait()
        bref = bufs[m % NBUF]

        @pl.loop(0, CHUNK)
        def _row(r, m=m, bref=bref):
            row = m * CHUNK + r
            cvec = plsc.load_gather(coef_v, [jnp.full((LANES,), row, jnp.int32)])
            prow = row // 4
            for j in range(D_VECS):
                sl = pl.ds(j * LANES, LANES)
                bref[r, sl] = bref[r, sl] * cvec + pe_v[prow, sl]

        pending_o[m] = issue_out(m)

    for k in sorted(pending_o):
        pending_o[k].wait()


@functools.partial(jax.jit, static_argnames=())
def kernel(inputs, table):
    s, b = inputs.shape
    idx_flat = inputs.reshape(-1).astype(jnp.int32)
    pe = _pe_table(s)

    mesh = plsc.VectorSubcoreMesh(
        core_axis_name="c", subcore_axis_name="s",
        num_cores=NC, num_subcores=NS)

    run = pl.kernel(
        _sc_body,
        out_type=jax.ShapeDtypeStruct((N_ROWS, D_MODEL), jnp.float32),
        mesh=mesh,
        compiler_params=pltpu.CompilerParams(needs_layout_passes=False),
        scratch_types=[
            pltpu.VMEM((R_PER_W,), jnp.int32),
            pltpu.VMEM((R_PER_W,), jnp.float32),
            pltpu.VMEM((PE_PER_W, D_MODEL), jnp.float32),
            pltpu.VMEM((CHUNK, D_MODEL), jnp.float32),
            pltpu.VMEM((CHUNK, D_MODEL), jnp.float32),
            pltpu.VMEM((CHUNK, D_MODEL), jnp.float32),
            pltpu.SemaphoreType.DMA,
            pltpu.SemaphoreType.DMA,
            pltpu.SemaphoreType.DMA,
            pltpu.SemaphoreType.DMA,
            pltpu.SemaphoreType.DMA,
            pltpu.SemaphoreType.DMA,
        ],
    )
    out = run(table, idx_flat, pe)
    return out.reshape(s, b, D_MODEL)
